# Initial kernel scaffold; baseline (speedup 1.0000x reference)
import jax
import jax.numpy as jnp
from jax.experimental import pallas as pl


def kernel(x, edge_index, initial_weight, W_ih, W_hh, b_ih, b_hh):
    raise NotImplementedError("write your pallas kernel here")


# speedup vs baseline: 25.2139x; 25.2139x over previous
import functools

import jax
import jax.numpy as jnp
from jax import lax
from jax.experimental import pallas as pl
from jax.experimental.pallas import tpu as pltpu
from jax.experimental.pallas import tpu_sc as plsc

N_NODES = 10000
N_EDGES = 320000
HIDDEN = 128

NC = 2
NS = 16
NW = NC * NS
NPAD = 10112
RPT = NPAD // NS
CHUNK = 128
NCH = 80
EPW = NCH * CHUNK
EPAD = NW * EPW

_mesh = plsc.VectorSubcoreMesh(
    core_axis_name="c", subcore_axis_name="s", num_cores=NC, num_subcores=NS)


@functools.partial(
    pl.kernel,
    out_type=jax.ShapeDtypeStruct((NC, NPAD, HIDDEN), jnp.float32),
    mesh=_mesh,
    scratch_types=[
        pltpu.VMEM((NCH, CHUNK), jnp.int32),
        pltpu.VMEM((CHUNK, HIDDEN), jnp.float32),
        pltpu.VMEM_SHARED((NPAD, HIDDEN), jnp.float32),
    ],
)
def _deg_kernel(dst_hbm, zeros_hbm, ones_hbm, out_hbm, idx_v, ones_v, deg_sh):
    c = lax.axis_index("c")
    s = lax.axis_index("s")
    wid = c * NS + s
    pltpu.sync_copy(dst_hbm.at[wid], idx_v)
    pltpu.sync_copy(ones_hbm, ones_v)
    pltpu.sync_copy(zeros_hbm.at[pl.ds(s * RPT, RPT)],
                    deg_sh.at[pl.ds(s * RPT, RPT)])
    plsc.subcore_barrier()

    def body(j, carry):
        pltpu.sync_copy(ones_v, deg_sh.at[idx_v.at[j]], add=True)
        return carry

    lax.fori_loop(0, NCH, body, 0)
    plsc.subcore_barrier()
    pltpu.sync_copy(deg_sh.at[pl.ds(s * RPT, RPT)],
                    out_hbm.at[c, pl.ds(s * RPT, RPT)])


@functools.partial(
    pl.kernel,
    out_type=jax.ShapeDtypeStruct((NC, NPAD, HIDDEN), jnp.float32),
    mesh=_mesh,
    scratch_types=[
        pltpu.VMEM((NCH, CHUNK), jnp.int32),
        pltpu.VMEM((NCH, CHUNK), jnp.int32),
        pltpu.VMEM((CHUNK, HIDDEN), jnp.float32),
        pltpu.VMEM_SHARED((NPAD, HIDDEN), jnp.float32),
        pltpu.SemaphoreType.DMA,
    ],
)
def _agg_kernel(src_hbm, dst_hbm, y_hbm, zeros_hbm, out_hbm,
                src_v, dst_v, gbuf, acc_sh, sem):
    c = lax.axis_index("c")
    s = lax.axis_index("s")
    wid = c * NS + s
    pltpu.sync_copy(src_hbm.at[wid], src_v)
    pltpu.sync_copy(dst_hbm.at[wid], dst_v)
    pltpu.sync_copy(zeros_hbm.at[pl.ds(s * RPT, RPT)],
                    acc_sh.at[pl.ds(s * RPT, RPT)])
    plsc.subcore_barrier()

    def body(j, carry):
        pltpu.async_copy(y_hbm.at[src_v.at[j]], gbuf, sem).wait()
        pltpu.sync_copy(gbuf, acc_sh.at[dst_v.at[j]], add=True)
        return carry

    lax.fori_loop(0, NCH, body, 0)
    plsc.subcore_barrier()
    pltpu.sync_copy(acc_sh.at[pl.ds(s * RPT, RPT)],
                    out_hbm.at[c, pl.ds(s * RPT, RPT)])


def _project_body(x_ref, hist_ref, w0_ref, wih_ref, b_ref, y_ref, dinv_ref):
    gates = lax.dot_general(
        w0_ref[...], wih_ref[...], (((1,), (1,)), ((), ())),
        preferred_element_type=jnp.float32) + b_ref[...]
    i = jax.nn.sigmoid(gates[:, 0 * HIDDEN:1 * HIDDEN])
    g = jnp.tanh(gates[:, 2 * HIDDEN:3 * HIDDEN])
    o = jax.nn.sigmoid(gates[:, 3 * HIDDEN:4 * HIDDEN])
    w_new = o * jnp.tanh(i * g)
    deg = hist_ref[0, :, 0:1] + hist_ref[1, :, 0:1] + 1.0
    dinv = lax.rsqrt(deg)
    y_ref[...] = jnp.dot(x_ref[...], w_new,
                         preferred_element_type=jnp.float32) * dinv
    dinv_ref[...] = dinv


_project = pl.pallas_call(
    _project_body,
    out_shape=(jax.ShapeDtypeStruct((NPAD, HIDDEN), jnp.float32),
               jax.ShapeDtypeStruct((NPAD, 1), jnp.float32)),
)


def _combine_body(acc_ref, y_ref, dinv_ref, out_ref):
    out_ref[...] = (acc_ref[0, :N_NODES, :] + acc_ref[1, :N_NODES, :]
                    + y_ref[:N_NODES, :]) * dinv_ref[:N_NODES, :]


_combine = pl.pallas_call(
    _combine_body,
    out_shape=jax.ShapeDtypeStruct((N_NODES, HIDDEN), jnp.float32),
)


def kernel(x, edge_index, initial_weight, W_ih, W_hh, b_ih, b_hh):
    del W_hh
    src = edge_index[0]
    dst = edge_index[1]
    npad_edges = EPAD - N_EDGES
    fill = N_NODES + (jnp.arange(npad_edges, dtype=jnp.int32) % (NPAD - N_NODES))
    src_p = jnp.concatenate([src, fill]).reshape(NW, NCH, CHUNK)
    dst_p = jnp.concatenate([dst, fill]).reshape(NW, NCH, CHUNK)
    x_pad = jnp.zeros((NPAD, HIDDEN), jnp.float32).at[:N_NODES].set(x)

    zeros_acc = jnp.zeros((NPAD, HIDDEN), jnp.float32)
    ones_blk = jnp.ones((CHUNK, HIDDEN), jnp.float32)

    hist = _deg_kernel(dst_p, zeros_acc, ones_blk)
    hist_col = hist[:, :, :1]
    b = (b_ih + b_hh).reshape(1, 4 * HIDDEN)
    y, dinv = _project(x_pad, hist_col, initial_weight, W_ih, b)
    accs = _agg_kernel(src_p, dst_p, y, zeros_acc)
    return _combine(accs, y, dinv)


# speedup vs baseline: 29.4265x; 1.1671x over previous
import functools

import jax
import jax.numpy as jnp
from jax import lax
from jax.experimental import pallas as pl
from jax.experimental.pallas import tpu as pltpu
from jax.experimental.pallas import tpu_sc as plsc

N_NODES = 10000
N_EDGES = 320000
HIDDEN = 128

NC = 2
NS = 16
NW = NC * NS
NPAD = 10112
RPT = NPAD // NS
CHUNK = 128
NCH = 80
HALF = NCH // 2
EPW = NCH * CHUNK
EPAD = NW * EPW

_mesh = plsc.VectorSubcoreMesh(
    core_axis_name="c", subcore_axis_name="s", num_cores=NC, num_subcores=NS)


@functools.partial(
    pl.kernel,
    out_type=jax.ShapeDtypeStruct((NC, NPAD, HIDDEN), jnp.float32),
    mesh=_mesh,
    scratch_types=[
        pltpu.VMEM((NCH, CHUNK), jnp.int32),
        pltpu.VMEM((CHUNK, HIDDEN), jnp.float32),
        pltpu.VMEM_SHARED((NPAD, HIDDEN), jnp.float32),
        pltpu.SemaphoreType.DMA,
    ],
)
def _deg_kernel(dst_hbm, zeros_hbm, ones_hbm, out_hbm, idx_v, ones_v, deg_sh,
                sem):
    c = lax.axis_index("c")
    s = lax.axis_index("s")
    wid = c * NS + s
    pltpu.sync_copy(dst_hbm.at[wid], idx_v)
    pltpu.sync_copy(ones_hbm, ones_v)
    pltpu.sync_copy(zeros_hbm.at[pl.ds(s * RPT, RPT)],
                    deg_sh.at[pl.ds(s * RPT, RPT)])
    plsc.subcore_barrier()

    K = 8

    def body(g, carry):
        descs = [
            pltpu.async_copy(ones_v, deg_sh.at[idx_v.at[g * K + k]], sem,
                             add=True)
            for k in range(K)
        ]
        for d in descs:
            d.wait()
        return carry

    lax.fori_loop(0, NCH // K, body, 0)
    plsc.subcore_barrier()
    pltpu.sync_copy(deg_sh.at[pl.ds(s * RPT, RPT)],
                    out_hbm.at[c, pl.ds(s * RPT, RPT)])


@functools.partial(
    pl.kernel,
    out_type=jax.ShapeDtypeStruct((NC, NPAD, HIDDEN), jnp.float32),
    mesh=_mesh,
    scratch_types=[
        pltpu.VMEM((HALF, CHUNK), jnp.int32),
        pltpu.VMEM((HALF, CHUNK), jnp.int32),
        pltpu.VMEM((CHUNK, HIDDEN), jnp.float32),
        pltpu.VMEM((CHUNK, HIDDEN), jnp.float32),
        pltpu.VMEM_SHARED((NPAD, HIDDEN), jnp.float32),
        pltpu.SemaphoreType.DMA,
        pltpu.SemaphoreType.DMA,
    ],
)
def _agg_kernel(src_hbm, dst_hbm, y_hbm, zeros_hbm, out_hbm,
                src_v, dst_v, gbuf0, gbuf1, acc_sh, sem0, sem1):
    c = lax.axis_index("c")
    s = lax.axis_index("s")
    wid = c * NS + s
    pltpu.sync_copy(zeros_hbm.at[pl.ds(s * RPT, RPT)],
                    acc_sh.at[pl.ds(s * RPT, RPT)])
    plsc.subcore_barrier()

    def half_body(h, carry):
        row0 = pl.multiple_of(h * HALF, 8)
        pltpu.sync_copy(src_hbm.at[wid, pl.ds(row0, HALF)], src_v)
        pltpu.sync_copy(dst_hbm.at[wid, pl.ds(row0, HALF)], dst_v)
        pltpu.async_copy(y_hbm.at[src_v.at[0]], gbuf0, sem0)

        def body(p, carry2):
            c0 = 2 * p
            pltpu.make_async_copy(y_hbm.at[src_v.at[c0]], gbuf0, sem0).wait()
            pltpu.async_copy(y_hbm.at[src_v.at[c0 + 1]], gbuf1, sem1)
            pltpu.sync_copy(gbuf0, acc_sh.at[dst_v.at[c0]], add=True)
            pltpu.make_async_copy(y_hbm.at[src_v.at[c0 + 1]], gbuf1,
                                  sem1).wait()
            nxt = jnp.minimum(c0 + 2, HALF - 1)
            pltpu.async_copy(y_hbm.at[src_v.at[nxt]], gbuf0, sem0)
            pltpu.sync_copy(gbuf1, acc_sh.at[dst_v.at[c0 + 1]], add=True)
            return carry2

        lax.fori_loop(0, HALF // 2, body, 0)
        pltpu.make_async_copy(y_hbm.at[src_v.at[HALF - 1]], gbuf0,
                              sem0).wait()
        return carry

    lax.fori_loop(0, 2, half_body, 0)
    plsc.subcore_barrier()
    pltpu.sync_copy(acc_sh.at[pl.ds(s * RPT, RPT)],
                    out_hbm.at[c, pl.ds(s * RPT, RPT)])


def _evolve_body(x_ref, w0_ref, wih_ref, b_ref, xw_ref):
    gates = lax.dot_general(
        w0_ref[...], wih_ref[...], (((1,), (1,)), ((), ())),
        preferred_element_type=jnp.float32) + b_ref[...]
    i = jax.nn.sigmoid(gates[:, 0 * HIDDEN:1 * HIDDEN])
    g = jnp.tanh(gates[:, 2 * HIDDEN:3 * HIDDEN])
    o = jax.nn.sigmoid(gates[:, 3 * HIDDEN:4 * HIDDEN])
    w_new = o * jnp.tanh(i * g)
    xw_ref[...] = jnp.dot(x_ref[...], w_new,
                          preferred_element_type=jnp.float32)


_evolve = pl.pallas_call(
    _evolve_body,
    out_shape=jax.ShapeDtypeStruct((NPAD, HIDDEN), jnp.float32),
)


def _scale_body(xw_ref, hist_ref, y_ref, dinv_ref):
    deg = hist_ref[0, :, 0:1] + hist_ref[1, :, 0:1] + 1.0
    dinv = lax.rsqrt(deg)
    y_ref[...] = xw_ref[...] * dinv
    dinv_ref[...] = dinv


_scale = pl.pallas_call(
    _scale_body,
    out_shape=(jax.ShapeDtypeStruct((NPAD, HIDDEN), jnp.float32),
               jax.ShapeDtypeStruct((NPAD, 1), jnp.float32)),
)


def _combine_body(acc_ref, y_ref, dinv_ref, out_ref):
    out_ref[...] = (acc_ref[0, :N_NODES, :] + acc_ref[1, :N_NODES, :]
                    + y_ref[:N_NODES, :]) * dinv_ref[:N_NODES, :]


_combine = pl.pallas_call(
    _combine_body,
    out_shape=jax.ShapeDtypeStruct((N_NODES, HIDDEN), jnp.float32),
)


def kernel(x, edge_index, initial_weight, W_ih, W_hh, b_ih, b_hh):
    del W_hh
    src = edge_index[0]
    dst = edge_index[1]
    npad_edges = EPAD - N_EDGES
    fill = N_NODES + (jnp.arange(npad_edges, dtype=jnp.int32) % (NPAD - N_NODES))
    src_p = jnp.concatenate([src, fill]).reshape(NW, NCH, CHUNK)
    dst_p = jnp.concatenate([dst, fill]).reshape(NW, NCH, CHUNK)
    x_pad = jnp.zeros((NPAD, HIDDEN), jnp.float32).at[:N_NODES].set(x)

    zeros_acc = jnp.zeros((NPAD, HIDDEN), jnp.float32)
    ones_blk = jnp.ones((CHUNK, HIDDEN), jnp.float32)

    hist = _deg_kernel(dst_p, zeros_acc, ones_blk)
    hist_col = hist[:, :, :1]
    b = (b_ih + b_hh).reshape(1, 4 * HIDDEN)
    xw = _evolve(x_pad, initial_weight, W_ih, b)
    y, dinv = _scale(xw, hist_col)
    accs = _agg_kernel(src_p, dst_p, y, zeros_acc)
    return _combine(accs, y, dinv)
